# baseline (speedup 1.0000x reference)
import functools

import jax
import jax.numpy as jnp
from jax import lax
from jax.experimental import pallas as pl
from jax.experimental.pallas import tpu as pltpu
from jax.experimental.pallas import tpu_sc as plsc

_N = 10000
_E = 160000
_F = 32
_DE = 4
_H = 64

_NW = 32
_IDXW = 128
_ROWS_W = 40
_EW = _ROWS_W * _IDXW
_EPAD = _NW * _EW
_BLK = 8
_ECH = _BLK * _IDXW
_NCH = _ROWS_W // _BLK

_NPAD = 10112
_DUMMY = 10000
_STRIPE = _NPAD // 16


def _sc_mesh():
    return plsc.VectorSubcoreMesh(core_axis_name="c", subcore_axis_name="s")


def _sc_gather(table, idx2d):
    @functools.partial(
        pl.kernel,
        mesh=_sc_mesh(),
        out_type=jax.ShapeDtypeStruct((_EPAD, _F), jnp.bfloat16),
        scratch_types=[
            pltpu.VMEM((2, _BLK, _IDXW), jnp.int32),
            pltpu.VMEM((2, _ECH, _F), jnp.bfloat16),
            pltpu.SemaphoreType.DMA,
            pltpu.SemaphoreType.DMA,
            pltpu.SemaphoreType.DMA,
        ],
        compiler_params=pltpu.CompilerParams(use_tc_tiling_on_sc=False),
    )
    def k(table_hbm, idx_hbm, out_hbm, idx_v, rows_v, gsem, isem, osem):
        wid = lax.axis_index("s") * 2 + lax.axis_index("c")
        row0 = wid * _ROWS_W
        pltpu.sync_copy(idx_hbm.at[pl.ds(row0, _BLK)], idx_v.at[0])
        stores = {}
        for j in range(_NCH):
            cur = j & 1
            if j >= 2:
                stores[j - 2].wait()
            gs = [
                pltpu.async_copy(
                    table_hbm.at[idx_v.at[cur].at[b]],
                    rows_v.at[cur].at[pl.ds(b * _IDXW, _IDXW)],
                    gsem,
                )
                for b in range(_BLK)
            ]
            pre = None
            if j + 1 < _NCH:
                pre = pltpu.async_copy(
                    idx_hbm.at[pl.ds(row0 + (j + 1) * _BLK, _BLK)],
                    idx_v.at[1 - cur],
                    isem,
                )
            for g in gs:
                g.wait()
            stores[j] = pltpu.async_copy(
                rows_v.at[cur],
                out_hbm.at[pl.ds(wid * _EW + j * _ECH, _ECH)],
                osem,
            )
            if pre is not None:
                pre.wait()
        stores[_NCH - 2].wait()
        stores[_NCH - 1].wait()

    return k(table, idx2d)


def _sc_scatter(msg, dst2d, z_nodes, z_cnt, ones_cnt, with_cnt):
    outs = [jax.ShapeDtypeStruct((2, _NPAD, _F), jnp.float32)]
    scratch = [
        pltpu.VMEM((2, _BLK, _IDXW), jnp.int32),
        pltpu.VMEM((2, _ECH, _F), jnp.float32),
        pltpu.VMEM_SHARED((_NPAD, _F), jnp.float32),
        pltpu.SemaphoreType.DMA,
        pltpu.SemaphoreType.DMA,
    ]
    if with_cnt:
        outs.append(jax.ShapeDtypeStruct((2, _NPAD, 16), jnp.float32))
        scratch.append(pltpu.VMEM((_IDXW, 16), jnp.float32))
        scratch.append(pltpu.VMEM_SHARED((_NPAD, 16), jnp.float32))

    @functools.partial(
        pl.kernel,
        mesh=_sc_mesh(),
        out_type=tuple(outs) if with_cnt else outs[0],
        scratch_types=scratch,
        compiler_params=pltpu.CompilerParams(use_tc_tiling_on_sc=False),
    )
    def k(msg_hbm, dst_hbm, zn_hbm, zc_hbm, ones_hbm, *refs):
        if with_cnt:
            out_s, out_c, idx_v, msg_v, acc, asem, lsem, ones_v, cacc = refs
        else:
            out_s, idx_v, msg_v, acc, asem, lsem = refs
        cid = lax.axis_index("c")
        sid = lax.axis_index("s")
        wid = sid * 2 + cid
        stripe = pl.ds(sid * _STRIPE, _STRIPE)
        pltpu.sync_copy(zn_hbm, acc.at[stripe])
        if with_cnt:
            pltpu.sync_copy(zc_hbm, cacc.at[stripe])
            pltpu.sync_copy(ones_hbm, ones_v)
        plsc.subcore_barrier()
        row0 = wid * _ROWS_W
        pltpu.sync_copy(dst_hbm.at[pl.ds(row0, _BLK)], idx_v.at[0])
        pltpu.sync_copy(msg_hbm.at[pl.ds(wid * _EW, _ECH)], msg_v.at[0])
        for j in range(_NCH):
            cur = j & 1
            adds = [
                pltpu.async_copy(
                    msg_v.at[cur].at[pl.ds(b * _IDXW, _IDXW)],
                    acc.at[idx_v.at[cur].at[b]],
                    asem,
                    add=True,
                )
                for b in range(_BLK)
            ]
            if with_cnt:
                adds += [
                    pltpu.async_copy(
                        ones_v, cacc.at[idx_v.at[cur].at[b]], asem, add=True
                    )
                    for b in range(_BLK)
                ]
            pres = []
            if j + 1 < _NCH:
                pres.append(pltpu.async_copy(
                    dst_hbm.at[pl.ds(row0 + (j + 1) * _BLK, _BLK)],
                    idx_v.at[1 - cur],
                    lsem,
                ))
                pres.append(pltpu.async_copy(
                    msg_hbm.at[pl.ds(wid * _EW + (j + 1) * _ECH, _ECH)],
                    msg_v.at[1 - cur],
                    lsem,
                ))
            for a in adds:
                a.wait()
            for p in pres:
                p.wait()
        plsc.subcore_barrier()
        pltpu.sync_copy(acc.at[stripe], out_s.at[cid, stripe])
        if with_cnt:
            pltpu.sync_copy(cacc.at[stripe], out_c.at[cid, stripe])

    return k(msg, dst2d, z_nodes, z_cnt, ones_cnt)


_BE = 8192


_EQ = _EPAD // 4


def _tc_msg(ea3, xsp, w1t, b1, w2t, b2, w3t, smat):
    grid = (_EQ // _BE,)

    def body(ea_ref, xs_ref, w1_ref, b1_ref, w2_ref, b2_ref, w3_ref,
             s_ref, out_ref):
        outs = []
        for q in range(4):
            ht = jnp.dot(w1_ref[...], ea_ref[q],
                         preferred_element_type=jnp.float32) + b1_ref[...]
            ht = jnp.maximum(ht, 0.0)
            ht = jnp.dot(w2_ref[...], ht.astype(jnp.bfloat16),
                         preferred_element_type=jnp.float32) + b2_ref[...]
            ht = jnp.maximum(ht, 0.0)
            tht = jnp.dot(w3_ref[...], ht.astype(jnp.bfloat16),
                          preferred_element_type=jnp.float32)
            xst = jnp.transpose(xs_ref[:, q * _F:(q + 1) * _F])
            xstt = jnp.concatenate([xst] * _F, axis=0)
            msgt = jnp.dot(s_ref[...], (tht * xstt).astype(jnp.bfloat16),
                           preferred_element_type=jnp.float32)
            outs.append(jnp.transpose(msgt))
        out_ref[...] = jnp.concatenate(outs, axis=1)

    return pl.pallas_call(
        body,
        grid=grid,
        in_specs=[
            pl.BlockSpec((4, _DE, _BE), lambda i: (0, 0, i)),
            pl.BlockSpec((_BE, 4 * _F), lambda i: (i, 0)),
            pl.BlockSpec((_H, _DE), lambda i: (0, 0)),
            pl.BlockSpec((_H, 1), lambda i: (0, 0)),
            pl.BlockSpec((_H, _H), lambda i: (0, 0)),
            pl.BlockSpec((_H, 1), lambda i: (0, 0)),
            pl.BlockSpec((_F * _F, _H), lambda i: (0, 0)),
            pl.BlockSpec((_F, _F * _F), lambda i: (0, 0)),
        ],
        out_specs=pl.BlockSpec((_BE, 4 * _F), lambda i: (i, 0)),
        out_shape=jax.ShapeDtypeStruct((_EQ, 4 * _F), jnp.float32),
    )(ea3, xsp, w1t, b1, w2t, b2, w3t, smat)


def _tc_finalize(psums, pcnt, gamma, beta, resid):
    n_in = 4 if resid is None else 5

    def body(*refs):
        if resid is None:
            ps_ref, pc_ref, g_ref, b_ref, out_ref = refs
        else:
            ps_ref, pc_ref, g_ref, b_ref, r_ref, out_ref = refs
        s = ps_ref[0, 0:_N, :] + ps_ref[1, 0:_N, :]
        c = pc_ref[0, 0:_N, 0:1] + pc_ref[1, 0:_N, 0:1]
        h = s / jnp.maximum(c, 1.0)
        mu = jnp.mean(h, axis=0, keepdims=True)
        xc = h - mu
        var = jnp.mean(xc * xc, axis=0, keepdims=True)
        y = xc * lax.rsqrt(var + 1e-5) * g_ref[...] + b_ref[...]
        if resid is not None:
            y = y + r_ref[...]
        out_ref[...] = jnp.maximum(y, 0.0)

    args = [psums, pcnt, gamma.reshape(1, _F), beta.reshape(1, _F)]
    if resid is not None:
        args.append(resid)
    return pl.pallas_call(
        body,
        out_shape=jax.ShapeDtypeStruct((_N, _F), jnp.float32),
    )(*args)


def _prep_w3(w3):
    return w3.reshape(_F, _F, _H).transpose(1, 0, 2).reshape(_F * _F, _H).T


def kernel(x, edge_index, edge_attr, f1_W1, f1_b1, f1_W2, f1_b2, f1_W3,
           f2_W1, f2_b1, f2_W2, f2_b2, f2_W3, bn1_gamma, bn1_beta,
           bn2_gamma, bn2_beta):
    src = edge_index[0]
    dst = edge_index[1]
    npad = _EPAD - _E
    src2d = jnp.concatenate(
        [src, jnp.zeros((npad,), jnp.int32)]).reshape(
            4, _EQ).T.reshape(-1, _IDXW)
    dst2d = jnp.concatenate(
        [dst, jnp.full((npad,), _DUMMY, jnp.int32)]).reshape(
            4, _EQ).T.reshape(-1, _IDXW)
    ea3 = jnp.concatenate(
        [edge_attr, jnp.zeros((npad, _DE), jnp.float32)],
        axis=0).reshape(4, _EQ, _DE).transpose(0, 2, 1)

    z_nodes = jnp.zeros((_STRIPE, _F), jnp.float32)
    z_cnt = jnp.zeros((_STRIPE, 16), jnp.float32)
    ones_cnt = jnp.ones((_IDXW, 16), jnp.float32)
    bf = jnp.bfloat16
    smat_t = jnp.repeat(jnp.eye(_F, dtype=bf), _F, axis=0).T

    xs1 = _sc_gather(x.astype(bf), src2d).reshape(_EQ, 4 * _F)
    msg1 = _tc_msg(ea3, xs1, f1_W1, f1_b1.reshape(_H, 1),
                   f1_W2.astype(bf), f1_b2.reshape(_H, 1),
                   _prep_w3(f1_W3).T.astype(bf), smat_t).reshape(_EPAD, _F)
    ps1, pc = _sc_scatter(msg1, dst2d, z_nodes, z_cnt, ones_cnt, True)
    h = _tc_finalize(ps1, pc, bn1_gamma, bn1_beta, None)

    xs2 = _sc_gather(h.astype(bf), src2d).reshape(_EQ, 4 * _F)
    msg2 = _tc_msg(ea3, xs2, f2_W1, f2_b1.reshape(_H, 1),
                   f2_W2.astype(bf), f2_b2.reshape(_H, 1),
                   _prep_w3(f2_W3).T.astype(bf), smat_t).reshape(_EPAD, _F)
    ps2 = _sc_scatter(msg2, dst2d, z_nodes, z_cnt, ones_cnt, False)
    return _tc_finalize(ps2, pc, bn2_gamma, bn2_beta, x)


# speedup vs baseline: 1.0401x; 1.0401x over previous
import functools

import jax
import jax.numpy as jnp
from jax import lax
from jax.experimental import pallas as pl
from jax.experimental.pallas import tpu as pltpu
from jax.experimental.pallas import tpu_sc as plsc

_N = 10000
_E = 160000
_F = 32
_DE = 4
_H = 64

_NW = 32
_IDXW = 128
_ROWS_W = 40
_EW = _ROWS_W * _IDXW
_EPAD = _NW * _EW
_BLK = 8
_ECH = _BLK * _IDXW
_NCH = _ROWS_W // _BLK

_NPAD = 10112
_DUMMY = 10000
_STRIPE = _NPAD // 16


def _sc_mesh():
    return plsc.VectorSubcoreMesh(core_axis_name="c", subcore_axis_name="s")


def _sc_gather(table, idx2d):
    @functools.partial(
        pl.kernel,
        mesh=_sc_mesh(),
        out_type=jax.ShapeDtypeStruct((_EPAD, _F), jnp.bfloat16),
        scratch_types=[
            pltpu.VMEM((2, _BLK, _IDXW), jnp.int32),
            pltpu.VMEM((2, _ECH, _F), jnp.bfloat16),
            pltpu.SemaphoreType.DMA,
            pltpu.SemaphoreType.DMA,
            pltpu.SemaphoreType.DMA,
        ],
        compiler_params=pltpu.CompilerParams(use_tc_tiling_on_sc=False),
    )
    def k(table_hbm, idx_hbm, out_hbm, idx_v, rows_v, gsem, isem, osem):
        wid = lax.axis_index("s") * 2 + lax.axis_index("c")
        row0 = wid * _ROWS_W
        pltpu.sync_copy(idx_hbm.at[pl.ds(row0, _BLK)], idx_v.at[0])
        stores = {}
        for j in range(_NCH):
            cur = j & 1
            if j >= 2:
                stores[j - 2].wait()
            gs = [
                pltpu.async_copy(
                    table_hbm.at[idx_v.at[cur].at[b]],
                    rows_v.at[cur].at[pl.ds(b * _IDXW, _IDXW)],
                    gsem,
                )
                for b in range(_BLK)
            ]
            pre = None
            if j + 1 < _NCH:
                pre = pltpu.async_copy(
                    idx_hbm.at[pl.ds(row0 + (j + 1) * _BLK, _BLK)],
                    idx_v.at[1 - cur],
                    isem,
                )
            for g in gs:
                g.wait()
            stores[j] = pltpu.async_copy(
                rows_v.at[cur],
                out_hbm.at[pl.ds(wid * _EW + j * _ECH, _ECH)],
                osem,
            )
            if pre is not None:
                pre.wait()
        stores[_NCH - 2].wait()
        stores[_NCH - 1].wait()

    return k(table, idx2d)


def _sc_scatter(msg, dst2d, z_nodes, z_cnt, ones_cnt, with_cnt):
    outs = [jax.ShapeDtypeStruct((2, _NPAD, _F), jnp.float32)]
    scratch = [
        pltpu.VMEM((2, _BLK, _IDXW), jnp.int32),
        pltpu.VMEM((2, _ECH, _F), jnp.float32),
        pltpu.VMEM_SHARED((_NPAD, _F), jnp.float32),
        pltpu.SemaphoreType.DMA,
        pltpu.SemaphoreType.DMA,
    ]
    if with_cnt:
        outs.append(jax.ShapeDtypeStruct((2, _NPAD, 16), jnp.float32))
        scratch.append(pltpu.VMEM((_IDXW, 16), jnp.float32))
        scratch.append(pltpu.VMEM_SHARED((_NPAD, 16), jnp.float32))

    @functools.partial(
        pl.kernel,
        mesh=_sc_mesh(),
        out_type=tuple(outs) if with_cnt else outs[0],
        scratch_types=scratch,
        compiler_params=pltpu.CompilerParams(use_tc_tiling_on_sc=False),
    )
    def k(msg_hbm, dst_hbm, zn_hbm, zc_hbm, ones_hbm, *refs):
        if with_cnt:
            out_s, out_c, idx_v, msg_v, acc, asem, lsem, ones_v, cacc = refs
        else:
            out_s, idx_v, msg_v, acc, asem, lsem = refs
        cid = lax.axis_index("c")
        sid = lax.axis_index("s")
        wid = sid * 2 + cid
        stripe = pl.ds(sid * _STRIPE, _STRIPE)
        pltpu.sync_copy(zn_hbm, acc.at[stripe])
        if with_cnt:
            pltpu.sync_copy(zc_hbm, cacc.at[stripe])
            pltpu.sync_copy(ones_hbm, ones_v)
        plsc.subcore_barrier()
        row0 = wid * _ROWS_W
        pltpu.sync_copy(dst_hbm.at[pl.ds(row0, _BLK)], idx_v.at[0])
        pltpu.sync_copy(msg_hbm.at[pl.ds(wid * _EW, _ECH)], msg_v.at[0])
        for j in range(_NCH):
            cur = j & 1
            adds = [
                pltpu.async_copy(
                    msg_v.at[cur].at[pl.ds(b * _IDXW, _IDXW)],
                    acc.at[idx_v.at[cur].at[b]],
                    asem,
                    add=True,
                )
                for b in range(_BLK)
            ]
            if with_cnt:
                adds += [
                    pltpu.async_copy(
                        ones_v, cacc.at[idx_v.at[cur].at[b]], asem, add=True
                    )
                    for b in range(_BLK)
                ]
            pres = []
            if j + 1 < _NCH:
                pres.append(pltpu.async_copy(
                    dst_hbm.at[pl.ds(row0 + (j + 1) * _BLK, _BLK)],
                    idx_v.at[1 - cur],
                    lsem,
                ))
                pres.append(pltpu.async_copy(
                    msg_hbm.at[pl.ds(wid * _EW + (j + 1) * _ECH, _ECH)],
                    msg_v.at[1 - cur],
                    lsem,
                ))
            for a in adds:
                a.wait()
            for p in pres:
                p.wait()
        plsc.subcore_barrier()
        pltpu.sync_copy(acc.at[stripe], out_s.at[cid, stripe])
        if with_cnt:
            pltpu.sync_copy(cacc.at[stripe], out_c.at[cid, stripe])

    return k(msg, dst2d, z_nodes, z_cnt, ones_cnt)


_BE = 4096


_EQ = _EPAD // 4


def _tc_msg(ea3, xsp, w1t, b1, w2t, b2, w3t, smat):
    grid = (_EQ // _BE,)

    def body(ea_ref, xs_ref, w1_ref, b1_ref, w2_ref, b2_ref, w3_ref,
             s_ref, out_ref):
        outs = []
        for q in range(4):
            ht = jnp.dot(w1_ref[...], ea_ref[q],
                         preferred_element_type=jnp.float32) + b1_ref[...]
            ht = jnp.maximum(ht, 0.0)
            ht = jnp.dot(w2_ref[...], ht.astype(jnp.bfloat16),
                         preferred_element_type=jnp.float32) + b2_ref[...]
            ht = jnp.maximum(ht, 0.0)
            tht = jnp.dot(w3_ref[...], ht.astype(jnp.bfloat16),
                          preferred_element_type=jnp.float32)
            xst = jnp.transpose(xs_ref[:, q * _F:(q + 1) * _F])
            xstt = jnp.concatenate([xst] * _F, axis=0)
            msgt = jnp.dot(s_ref[...], (tht * xstt).astype(jnp.bfloat16),
                           preferred_element_type=jnp.float32)
            outs.append(jnp.transpose(msgt))
        out_ref[...] = jnp.concatenate(outs, axis=1)

    return pl.pallas_call(
        body,
        grid=grid,
        in_specs=[
            pl.BlockSpec((4, _DE, _BE), lambda i: (0, 0, i)),
            pl.BlockSpec((_BE, 4 * _F), lambda i: (i, 0)),
            pl.BlockSpec((_H, _DE), lambda i: (0, 0)),
            pl.BlockSpec((_H, 1), lambda i: (0, 0)),
            pl.BlockSpec((_H, _H), lambda i: (0, 0)),
            pl.BlockSpec((_H, 1), lambda i: (0, 0)),
            pl.BlockSpec((_F * _F, _H), lambda i: (0, 0)),
            pl.BlockSpec((_F, _F * _F), lambda i: (0, 0)),
        ],
        out_specs=pl.BlockSpec((_BE, 4 * _F), lambda i: (i, 0)),
        out_shape=jax.ShapeDtypeStruct((_EQ, 4 * _F), jnp.float32),
    )(ea3, xsp, w1t, b1, w2t, b2, w3t, smat)


def _tc_finalize(psums, pcnt, gamma, beta, resid):
    n_in = 4 if resid is None else 5

    def body(*refs):
        if resid is None:
            ps_ref, pc_ref, g_ref, b_ref, out_ref = refs
        else:
            ps_ref, pc_ref, g_ref, b_ref, r_ref, out_ref = refs
        s = ps_ref[0, 0:_N, :] + ps_ref[1, 0:_N, :]
        c = pc_ref[0, 0:_N, 0:1] + pc_ref[1, 0:_N, 0:1]
        h = s / jnp.maximum(c, 1.0)
        mu = jnp.mean(h, axis=0, keepdims=True)
        xc = h - mu
        var = jnp.mean(xc * xc, axis=0, keepdims=True)
        y = xc * lax.rsqrt(var + 1e-5) * g_ref[...] + b_ref[...]
        if resid is not None:
            y = y + r_ref[...]
        out_ref[...] = jnp.maximum(y, 0.0)

    args = [psums, pcnt, gamma.reshape(1, _F), beta.reshape(1, _F)]
    if resid is not None:
        args.append(resid)
    return pl.pallas_call(
        body,
        out_shape=jax.ShapeDtypeStruct((_N, _F), jnp.float32),
    )(*args)


def _prep_w3(w3):
    return w3.reshape(_F, _F, _H).transpose(1, 0, 2).reshape(_F * _F, _H).T


def kernel(x, edge_index, edge_attr, f1_W1, f1_b1, f1_W2, f1_b2, f1_W3,
           f2_W1, f2_b1, f2_W2, f2_b2, f2_W3, bn1_gamma, bn1_beta,
           bn2_gamma, bn2_beta):
    src = edge_index[0]
    dst = edge_index[1]
    npad = _EPAD - _E
    src2d = jnp.concatenate(
        [src, jnp.zeros((npad,), jnp.int32)]).reshape(
            4, _EQ).T.reshape(-1, _IDXW)
    dst2d = jnp.concatenate(
        [dst, jnp.full((npad,), _DUMMY, jnp.int32)]).reshape(
            4, _EQ).T.reshape(-1, _IDXW)
    ea3 = jnp.concatenate(
        [edge_attr, jnp.zeros((npad, _DE), jnp.float32)],
        axis=0).reshape(4, _EQ, _DE).transpose(0, 2, 1)

    z_nodes = jnp.zeros((_STRIPE, _F), jnp.float32)
    z_cnt = jnp.zeros((_STRIPE, 16), jnp.float32)
    ones_cnt = jnp.ones((_IDXW, 16), jnp.float32)
    bf = jnp.bfloat16
    smat_t = jnp.repeat(jnp.eye(_F, dtype=bf), _F, axis=0).T

    xs1 = _sc_gather(x.astype(bf), src2d).reshape(_EQ, 4 * _F)
    msg1 = _tc_msg(ea3, xs1, f1_W1, f1_b1.reshape(_H, 1),
                   f1_W2.astype(bf), f1_b2.reshape(_H, 1),
                   _prep_w3(f1_W3).T.astype(bf), smat_t).reshape(_EPAD, _F)
    ps1, pc = _sc_scatter(msg1, dst2d, z_nodes, z_cnt, ones_cnt, True)
    h = _tc_finalize(ps1, pc, bn1_gamma, bn1_beta, None)

    xs2 = _sc_gather(h.astype(bf), src2d).reshape(_EQ, 4 * _F)
    msg2 = _tc_msg(ea3, xs2, f2_W1, f2_b1.reshape(_H, 1),
                   f2_W2.astype(bf), f2_b2.reshape(_H, 1),
                   _prep_w3(f2_W3).T.astype(bf), smat_t).reshape(_EPAD, _F)
    ps2 = _sc_scatter(msg2, dst2d, z_nodes, z_cnt, ones_cnt, False)
    return _tc_finalize(ps2, pc, bn2_gamma, bn2_beta, x)
